# baseline (speedup 1.0000x reference)
import functools

import jax
import jax.numpy as jnp
from jax import lax
from jax.experimental import pallas as pl
from jax.experimental.pallas import tpu as pltpu
from jax.experimental.pallas import tpu_sc as plsc

_INFO = plsc.get_sparse_core_info()
_NC = _INFO.num_cores
_NS = _INFO.num_subcores
_NW = _NC * _NS

_NBUF = 8
_GLA = 4


def _make_sc_call(b_total, l_total, emb_w, emb_f, chunk, n_chunks):
    d_out = emb_w + emb_f
    sent_per_w = b_total // _NW
    cps = l_total // chunk
    assert n_chunks == sent_per_w * cps and n_chunks % _NBUF == 0
    n_blocks = n_chunks // _NBUF
    mesh = plsc.VectorSubcoreMesh(core_axis_name="c", subcore_axis_name="s")

    scratch = [
        pltpu.VMEM((n_chunks, chunk), jnp.int32),
        pltpu.VMEM((n_chunks, chunk), jnp.int32),
    ]
    scratch += [pltpu.VMEM((chunk, emb_w), jnp.float32) for _ in range(_NBUF)]
    scratch += [pltpu.VMEM((chunk, emb_f), jnp.float32) for _ in range(_NBUF)]
    scratch += [pltpu.SemaphoreType.DMA for _ in range(2 * _NBUF)]

    @functools.partial(
        pl.kernel,
        out_type=jax.ShapeDtypeStruct((b_total, l_total, d_out), jnp.float32),
        mesh=mesh,
        scratch_types=scratch,
        compiler_params=pltpu.CompilerParams(use_tc_tiling_on_sc=False),
    )
    def sc_call(idx_w_hbm, idx_f_hbm, wtab_hbm, ftab_hbm, out_hbm, *refs):
        idx_w_v, idx_f_v = refs[0], refs[1]
        wbufs = refs[2:2 + _NBUF]
        fbufs = refs[2 + _NBUF:2 + 2 * _NBUF]
        gsems = refs[2 + 2 * _NBUF:2 + 3 * _NBUF]
        wsems = refs[2 + 3 * _NBUF:2 + 4 * _NBUF]

        wid = lax.axis_index("s") * _NC + lax.axis_index("c")
        sent0 = wid * sent_per_w
        pltpu.sync_copy(idx_w_hbm.at[wid], idx_w_v)
        pltpu.sync_copy(idx_f_hbm.at[wid], idx_f_v)

        def out_w(g):
            return out_hbm.at[sent0 + g // cps,
                              pl.ds((g % cps) * chunk, chunk), pl.ds(0, emb_w)]

        def out_f(g):
            return out_hbm.at[sent0 + g // cps,
                              pl.ds((g % cps) * chunk, chunk),
                              pl.ds(emb_w, emb_f)]

        def start_gathers(g, b):
            pltpu.async_copy(wtab_hbm.at[idx_w_v.at[g]], wbufs[b], gsems[b])
            pltpu.async_copy(ftab_hbm.at[idx_f_v.at[g]], fbufs[b], gsems[b])

        def wait_gathers(b):
            pltpu.make_async_copy(
                wtab_hbm.at[idx_w_v.at[0]], wbufs[b], gsems[b]).wait()
            pltpu.make_async_copy(
                ftab_hbm.at[idx_f_v.at[0]], fbufs[b], gsems[b]).wait()

        def start_writes(g, b):
            pltpu.async_copy(wbufs[b], out_w(g), wsems[b])
            pltpu.async_copy(fbufs[b], out_f(g), wsems[b])

        def wait_writes(g, b):
            pltpu.make_async_copy(wbufs[b], out_w(g), wsems[b]).wait()
            pltpu.make_async_copy(fbufs[b], out_f(g), wsems[b]).wait()

        def step(g, b, do_wait_write, do_gather):
            wait_gathers(b)
            start_writes(g, b)
            b2 = (b + _GLA) % _NBUF
            if do_wait_write:
                wait_writes(g + _GLA - _NBUF, b2)
            if do_gather:
                start_gathers(g + _GLA, b2)

        for b in range(_GLA):
            start_gathers(b, b)

        for b in range(_NBUF):
            step(b, b, do_wait_write=(b >= _NBUF - _GLA), do_gather=True)

        def body(blk, carry):
            g0 = blk * _NBUF
            for b in range(_NBUF):
                step(g0 + b, b, do_wait_write=True, do_gather=True)
            return carry

        lax.fori_loop(1, n_blocks - 1, body, 0)

        g0 = (n_blocks - 1) * _NBUF
        for b in range(_NBUF):
            g = g0 + b
            step(g, b, do_wait_write=True, do_gather=(g + _GLA < n_chunks))

        for g in range(n_chunks - _GLA, n_chunks):
            wait_writes(g, g % _NBUF)

    return sc_call


def kernel(word_inputs, feature_inputs, word_seq_lengths, word_table, feat_table):
    del word_seq_lengths
    b, l = word_inputs.shape
    vocab, emb_w = word_table.shape
    _, emb_f = feat_table.shape
    chunk = 100
    assert l % chunk == 0 and b % _NW == 0
    n_chunks = (b // _NW) * (l // chunk)

    idx_w = word_inputs.astype(jnp.int32).reshape(_NW, n_chunks, chunk)
    idx_f = feature_inputs[0].astype(jnp.int32).reshape(_NW, n_chunks, chunk)

    sc_call = _make_sc_call(b, l, emb_w, emb_f, chunk, n_chunks)
    return sc_call(idx_w, idx_f, word_table, feat_table)


# speedup vs baseline: 1.3087x; 1.3087x over previous
import functools

import jax
import jax.numpy as jnp
from jax import lax
from jax.experimental import pallas as pl
from jax.experimental.pallas import tpu as pltpu
from jax.experimental.pallas import tpu_sc as plsc

_INFO = plsc.get_sparse_core_info()
_NC = _INFO.num_cores
_NS = _INFO.num_subcores
_NL = _INFO.num_lanes
_NW = _NC * _NS

_CH = 40
_NBUF = 4
_GLA = 2


def _make_sc_call(b_total, l_total, emb_w, emb_f, feat_vocab, d_pad):
    d_out = emb_w + emb_f
    sent_per_w = b_total // _NW
    cps = l_total // _CH
    n_chunks = sent_per_w * cps
    n_per_w = sent_per_w * l_total
    assert n_chunks % _NBUF == 0 and n_chunks // _NBUF >= 3
    n_blocks = n_chunks // _NBUF
    mesh = plsc.VectorSubcoreMesh(core_axis_name="c", subcore_axis_name="s")

    scratch = [
        pltpu.VMEM((n_per_w,), jnp.int32),
        pltpu.VMEM((n_per_w + _NL,), jnp.int32),
        pltpu.VMEM((feat_vocab * emb_f,), jnp.float32),
    ]
    scratch += [pltpu.VMEM((_CH, d_pad), jnp.float32) for _ in range(_NBUF)]
    scratch += [pltpu.VMEM((_CH, d_out), jnp.float32) for _ in range(_NBUF)]
    scratch += [pltpu.SemaphoreType.DMA for _ in range(2 * _NBUF + 1)]

    @functools.partial(
        pl.kernel,
        out_type=jax.ShapeDtypeStruct((b_total, l_total, d_out), jnp.float32),
        mesh=mesh,
        scratch_types=scratch,
        compiler_params=pltpu.CompilerParams(needs_layout_passes=False),
    )
    def sc_call(idx_w_hbm, idx_f_hbm, wtab_hbm, ftab_hbm, out_hbm, *refs):
        idx_w_v, idx_f_v, ftab_v = refs[0], refs[1], refs[2]
        mbufs = refs[3:3 + _NBUF]
        wbufs = refs[3 + _NBUF:3 + 2 * _NBUF]
        gsems = refs[3 + 2 * _NBUF:3 + 3 * _NBUF]
        wsems = refs[3 + 3 * _NBUF:3 + 4 * _NBUF]
        ssem = refs[3 + 4 * _NBUF]

        wid = lax.axis_index("s") * _NC + lax.axis_index("c")
        sent0 = wid * sent_per_w
        pltpu.sync_copy(idx_w_hbm.at[wid], idx_w_v)
        pltpu.sync_copy(idx_f_hbm.at[wid], idx_f_v.at[pl.ds(0, n_per_w)])
        pltpu.async_copy(ftab_hbm, ftab_v, ssem).wait()

        lanes = lax.iota(jnp.int32, _NL)

        def out_slice(g):
            return out_hbm.at[sent0 + g // cps, pl.ds((g % cps) * _CH, _CH)]

        def start_gather(g, b):
            pltpu.async_copy(
                wtab_hbm.at[idx_w_v.at[pl.ds(g * _CH, _CH)]], mbufs[b],
                gsems[b])

        def wait_gather(b):
            pltpu.make_async_copy(
                wtab_hbm.at[idx_w_v.at[pl.ds(0, _CH)]], mbufs[b],
                gsems[b]).wait()

        def start_write(g, b):
            pltpu.async_copy(wbufs[b], out_slice(g), wsems[b])

        def wait_write(g, b):
            pltpu.make_async_copy(wbufs[b], out_slice(g), wsems[b]).wait()

        def merge(g, b):
            for r in range(_CH):
                for cb in range(emb_w // _NL):
                    wbufs[b][r, pl.ds(cb * _NL, _NL)] = (
                        mbufs[b][r, pl.ds(cb * _NL, _NL)])
            for r0 in range(0, _CH, _NL):
                cnt = min(_NL, _CH - r0)
                mask = None if cnt == _NL else lanes < cnt
                fivec = idx_f_v[pl.ds(g * _CH + r0, _NL)]
                if mask is not None:
                    fivec = jnp.where(mask, fivec, 0)
                fbase = fivec * emb_f
                rows = lanes + r0
                for c in range(emb_f):
                    vals = plsc.load_gather(ftab_v, [fbase + c], mask=mask)
                    plsc.store_scatter(
                        wbufs[b],
                        [rows, jnp.full((_NL,), emb_w + c, jnp.int32)],
                        vals, mask=mask)

        def step(g, b, do_wait_write, do_gather):
            wait_gather(b)
            merge(g, b)
            start_write(g, b)
            b2 = (b + _GLA) % _NBUF
            if do_wait_write:
                wait_write(g + _GLA - _NBUF, b2)
            if do_gather:
                start_gather(g + _GLA, b2)

        for b in range(_GLA):
            start_gather(b, b)

        for b in range(_NBUF):
            step(b, b, do_wait_write=(b >= _NBUF - _GLA), do_gather=True)

        def body(blk, carry):
            g0 = blk * _NBUF
            for b in range(_NBUF):
                step(g0 + b, b, do_wait_write=True, do_gather=True)
            return carry

        lax.fori_loop(1, n_blocks - 1, body, 0)

        g0 = (n_blocks - 1) * _NBUF
        for b in range(_NBUF):
            g = g0 + b
            step(g, b, do_wait_write=True, do_gather=(g + _GLA < n_chunks))

        for g in range(n_chunks - _GLA, n_chunks):
            wait_write(g, g % _NBUF)

    return sc_call


def kernel(word_inputs, feature_inputs, word_seq_lengths, word_table, feat_table):
    del word_seq_lengths
    b, l = word_inputs.shape
    vocab, emb_w = word_table.shape
    feat_vocab, emb_f = feat_table.shape
    assert l % _CH == 0 and b % _NW == 0
    d_pad = 128

    wtab128 = jnp.pad(word_table, ((0, 0), (0, d_pad - emb_w)))
    ftab_flat = feat_table.reshape(-1)
    idx_w = word_inputs.astype(jnp.int32).reshape(_NW, (b // _NW) * l)
    idx_f = feature_inputs[0].astype(jnp.int32).reshape(_NW, (b // _NW) * l)

    sc_call = _make_sc_call(b, l, emb_w, emb_f, feat_vocab, d_pad)
    return sc_call(idx_w, idx_f, wtab128, ftab_flat)
